# baseline (speedup 1.0000x reference)
import numpy as np
import jax
from jax import lax
import jax.numpy as jnp
from jax.experimental import pallas as pl
from jax.experimental.pallas import tpu as pltpu


def _masked_idx(c: int, ratio: float) -> list:
    mask = np.linspace(0, c * (1 - ratio), int(c * ratio))
    return [i for i in range(c) if i in mask]


def _make_body(cb, nmask):
    def _body(x_ref, n_ref, o_ref):
        x = x_ref[...]
        lane = lax.broadcasted_iota(jnp.int32, x.shape, 3)
        r = x
        for j in range(nmask):
            nj = n_ref[j][..., None]
            r = jnp.where(lane == j * cb, nj, r)
        o_ref[...] = r
    return _body


def kernel(input1, noise):
    b, c, h, w = input1.shape
    idx = _masked_idx(c, 0.5)
    nmask = len(idx)
    cb = c // nmask
    if idx != [j * cb for j in range(nmask)]:
        raise ValueError("masked channels not uniformly spaced")

    xt = jnp.transpose(input1, (0, 2, 3, 1))
    nz = noise.reshape(nmask, b, h, w)

    bb = 8
    out_t = pl.pallas_call(
        _make_body(cb, nmask),
        grid=(b // bb,),
        in_specs=[
            pl.BlockSpec((bb, h, w, c), lambda i: (i, 0, 0, 0)),
            pl.BlockSpec((nmask, bb, h, w), lambda i: (0, i, 0, 0)),
        ],
        out_specs=pl.BlockSpec((bb, h, w, c), lambda i: (i, 0, 0, 0)),
        out_shape=jax.ShapeDtypeStruct((b, h, w, c), jnp.float32),
        compiler_params=pltpu.CompilerParams(
            dimension_semantics=("parallel",)),
    )(xt, nz)
    return jnp.transpose(out_t, (0, 3, 1, 2))
